# baseline (device time: 419119 ns/iter reference)
import os

import jax
import jax.numpy as jnp
from jax import lax
from jax.experimental import pallas as pl
from jax.experimental.pallas import tpu as pltpu

_SKIP_COMM = os.environ.get("SKIP_COMM") == "1"

N_DEV = 4
OFFS = (0, 1, 3, 2)
INV_SLOT = {1: 2, 2: 3, 3: 1}
SEND_ORDER = (1, 3, 2)


def _to_bf16(x):
    m, k = x.shape
    blk = 512

    def body(x_ref, o_ref):
        o_ref[:, :] = x_ref[:, :].astype(jnp.bfloat16)

    return pl.pallas_call(
        body,
        grid=(m // blk,),
        out_shape=jax.ShapeDtypeStruct((m, k), jnp.bfloat16),
        in_specs=[pl.BlockSpec((blk, k), lambda i: (i, 0))],
        out_specs=pl.BlockSpec((blk, k), lambda i: (i, 0)),
    )(x)


def kernel(x, w_mat):
    m_total, k_blk = x.shape
    k_total, n_out = w_mat.shape
    m_blk = m_total // N_DEV
    n_tiles = 8
    n_tile = n_out // n_tiles
    m_sub = 4
    m_tile = m_blk // m_sub

    x16 = _to_bf16(x)

    def body(x_hbm, w_hbm, out_hbm, comm_ref, wbuf, abuf, obuf,
             send_sems, recv_sems, local_sem, w_sems, a_sems, o_sem):
        idx = pl.program_id(0)
        n = pl.program_id(1)
        my = lax.axis_index("i")
        lin = idx * n_tiles + n
        par = lin % 2
        last = N_DEV * n_tiles - 1

        def w_dma(i_, n_, p_):
            off_val = jnp.where(i_ == 1, 1, jnp.where(i_ == 2, 3,
                                                      jnp.where(i_ == 3, 2, 0)))
            jj = (my + off_val) % N_DEV
            return pltpu.make_async_copy(
                w_hbm.at[pl.ds(jj * k_blk, k_blk), pl.ds(n_ * n_tile, n_tile)],
                wbuf.at[p_],
                w_sems.at[p_],
            )

        def a_dma(n_, p_):
            return pltpu.make_async_copy(
                out_hbm.at[:, pl.ds(n_ * n_tile, n_tile)],
                abuf.at[p_],
                a_sems.at[p_],
            )

        def o_dma(n_):
            return pltpu.make_async_copy(
                obuf,
                out_hbm.at[:, pl.ds(n_ * n_tile, n_tile)],
                o_sem,
            )

        @pl.when(lin == 0)
        def _comm():
            if not _SKIP_COMM:
                barrier = pltpu.get_barrier_semaphore()
                for off in (1, 2, 3):
                    pl.semaphore_signal(
                        barrier, inc=1,
                        device_id=((my + off) % N_DEV,),
                        device_id_type=pl.DeviceIdType.MESH,
                    )
                pl.semaphore_wait(barrier, N_DEV - 1)

                for off in SEND_ORDER:
                    dst = (my + off) % N_DEV
                    slot = INV_SLOT[off]
                    rdma = pltpu.make_async_remote_copy(
                        src_ref=x_hbm.at[pl.ds(dst * m_blk, m_blk), :],
                        dst_ref=comm_ref.at[slot],
                        send_sem=send_sems.at[off - 1],
                        recv_sem=recv_sems.at[slot],
                        device_id=(dst,),
                        device_id_type=pl.DeviceIdType.MESH,
                    )
                    rdma.start()

            w_dma(0, 0, 0).start()
            local = pltpu.make_async_copy(
                x_hbm.at[pl.ds(my * m_blk, m_blk), :],
                comm_ref.at[0],
                local_sem,
            )
            local.start()
            local.wait()

        if not _SKIP_COMM:
            for k in range(1, N_DEV):
                @pl.when((idx == k) & (n == 0))
                def _wait_recv(k=k):
                    recv = pltpu.make_async_remote_copy(
                        src_ref=comm_ref.at[k],
                        dst_ref=comm_ref.at[k],
                        send_sem=send_sems.at[0],
                        recv_sem=recv_sems.at[k],
                        device_id=(my,),
                        device_id_type=pl.DeviceIdType.MESH,
                    )
                    recv.wait_recv()

        @pl.when(lin < last)
        def _prefetch():
            nlin = lin + 1
            nidx = nlin // n_tiles
            nn = nlin % n_tiles
            npar = nlin % 2
            w_dma(nidx, nn, npar).start()
            @pl.when(nidx > 0)
            def _():
                a_dma(nn, npar).start()

        @pl.when(lin >= 1)
        def _obuf_free():
            o_dma(n).wait()

        w_dma(idx, n, par).wait()
        @pl.when(idx > 0)
        def _await_acc():
            a_dma(n, par).wait()

        wtile = wbuf[par].astype(jnp.bfloat16)
        c = 0.7978845608028654
        for mi in range(m_sub):
            row = pl.ds(mi * m_tile, m_tile)
            part = jnp.dot(
                comm_ref[idx, row, :], wtile,
                preferred_element_type=jnp.float32,
            )
            @pl.when(idx == 0)
            def _init(row=row, part=part):
                obuf[row, :] = part
            @pl.when((idx > 0) & (idx < N_DEV - 1))
            def _acc(row=row, part=part):
                obuf[row, :] = abuf[par, row, :] + part
            @pl.when(idx == N_DEV - 1)
            def _fin(row=row, part=part):
                a = abuf[par, row, :] + part
                obuf[row, :] = 0.5 * a * (
                    1.0 + jnp.tanh(c * (a + 0.044715 * a * a * a))
                )

        o_dma(n).start()

        @pl.when(lin == last)
        def _drain():
            o_dma(n).wait()
            if not _SKIP_COMM:
                for off in (1, 2, 3):
                    dst = (my + off) % N_DEV
                    slot = INV_SLOT[off]
                    send = pltpu.make_async_remote_copy(
                        src_ref=x_hbm.at[pl.ds(dst * m_blk, m_blk), :],
                        dst_ref=comm_ref.at[slot],
                        send_sem=send_sems.at[off - 1],
                        recv_sem=recv_sems.at[slot],
                        device_id=(dst,),
                        device_id_type=pl.DeviceIdType.MESH,
                    )
                    send.wait_send()

    return pl.pallas_call(
        body,
        grid=(N_DEV, n_tiles),
        out_shape=jax.ShapeDtypeStruct((m_blk, n_out), jnp.float32),
        in_specs=[
            pl.BlockSpec(memory_space=pl.ANY),
            pl.BlockSpec(memory_space=pl.ANY),
        ],
        out_specs=pl.BlockSpec(memory_space=pl.ANY),
        scratch_shapes=[
            pltpu.VMEM((N_DEV, m_blk, k_blk), jnp.bfloat16),
            pltpu.VMEM((2, k_blk, n_tile), jnp.float32),
            pltpu.VMEM((2, m_blk, n_tile), jnp.float32),
            pltpu.VMEM((m_blk, n_tile), jnp.float32),
            pltpu.SemaphoreType.DMA((3,)),
            pltpu.SemaphoreType.DMA((N_DEV,)),
            pltpu.SemaphoreType.DMA,
            pltpu.SemaphoreType.DMA((2,)),
            pltpu.SemaphoreType.DMA((2,)),
            pltpu.SemaphoreType.DMA,
        ],
        compiler_params=pltpu.CompilerParams(
            dimension_semantics=("arbitrary", "arbitrary"),
            collective_id=None if _SKIP_COMM else 0,
            vmem_limit_bytes=100 * 1024 * 1024,
        ),
    )(x16, w_mat)


# device time: 386808 ns/iter; 1.0835x vs baseline; 1.0835x over previous
import os

import jax
import jax.numpy as jnp
from jax import lax
from jax.experimental import pallas as pl
from jax.experimental.pallas import tpu as pltpu

_SKIP_COMM = os.environ.get("SKIP_COMM") == "1"

N_DEV = 4
OFFS = (0, 1, 3, 2)
INV_SLOT = {1: 2, 2: 3, 3: 1}
SEND_ORDER = (1, 3, 2)


D_OFF = (1, 3, 2, 0)


def kernel(x, w_mat):
    m_total, k_blk = x.shape
    k_total, n_out = w_mat.shape
    m_blk = m_total // N_DEV
    n_tiles = 8
    n_tile = n_out // n_tiles
    m_sub = 4
    m_tile = m_blk // m_sub
    r_chunk = 256
    c_chunks = m_blk // r_chunk

    def body(x_hbm, w_hbm, out_hbm, x16_scr, comm_ref, wbuf, abuf, obuf,
             cbuf, stage, send_sems, recv_sems, conv_sem, w_sems, a_sems,
             o_sem):
        idx = pl.program_id(0)
        n = pl.program_id(1)
        my = lax.axis_index("i")
        lin = idx * n_tiles + n
        par = lin % 2
        last = N_DEV * n_tiles - 1

        def w_dma(i_, n_, p_):
            off_val = jnp.where(i_ == 1, 1, jnp.where(i_ == 2, 3,
                                                      jnp.where(i_ == 3, 2, 0)))
            jj = (my + off_val) % N_DEV
            return pltpu.make_async_copy(
                w_hbm.at[pl.ds(jj * k_blk, k_blk), pl.ds(n_ * n_tile, n_tile)],
                wbuf.at[p_],
                w_sems.at[p_],
            )

        def a_dma(n_, p_):
            return pltpu.make_async_copy(
                out_hbm.at[:, pl.ds(n_ * n_tile, n_tile)],
                abuf.at[p_],
                a_sems.at[p_],
            )

        def o_dma(n_):
            return pltpu.make_async_copy(
                obuf,
                out_hbm.at[:, pl.ds(n_ * n_tile, n_tile)],
                o_sem,
            )

        @pl.when(lin == 0)
        def _comm():
            if not _SKIP_COMM:
                barrier = pltpu.get_barrier_semaphore()
                for off in (1, 2, 3):
                    pl.semaphore_signal(
                        barrier, inc=1,
                        device_id=((my + off) % N_DEV,),
                        device_id_type=pl.DeviceIdType.MESH,
                    )
                pl.semaphore_wait(barrier, N_DEV - 1)

            def in_dma(g):
                d, c = divmod(g, c_chunks)
                dst = (my + D_OFF[d]) % N_DEV
                return pltpu.make_async_copy(
                    x_hbm.at[pl.ds(dst * m_blk + c * r_chunk, r_chunk), :],
                    cbuf.at[g % 2],
                    conv_sem.at[g % 2],
                )

            def conv_out_dma(g):
                d, c = divmod(g, c_chunks)
                return pltpu.make_async_copy(
                    stage,
                    x16_scr.at[d, pl.ds(c * r_chunk, r_chunk), :],
                    conv_sem.at[2],
                )

            in_dma(0).start()
            in_dma(1).start()
            pending = None
            for g in range(N_DEV * c_chunks):
                d, c = divmod(g, c_chunks)
                in_dma(g).wait()
                val = cbuf[g % 2].astype(jnp.bfloat16)
                if d < 3:
                    if pending is not None:
                        conv_out_dma(pending).wait()
                        pending = None
                    stage[:, :] = val
                    if g + 2 < N_DEV * c_chunks:
                        in_dma(g + 2).start()
                    conv_out_dma(g).start()
                    pending = g
                    if c == c_chunks - 1:
                        conv_out_dma(pending).wait()
                        pending = None
                        if not _SKIP_COMM:
                            off = D_OFF[d]
                            dstd = (my + off) % N_DEV
                            slot = INV_SLOT[off]
                            rdma = pltpu.make_async_remote_copy(
                                src_ref=x16_scr.at[d],
                                dst_ref=comm_ref.at[slot],
                                send_sem=send_sems.at[d],
                                recv_sem=recv_sems.at[slot],
                                device_id=(dstd,),
                                device_id_type=pl.DeviceIdType.MESH,
                            )
                            rdma.start()
                else:
                    comm_ref[0, pl.ds(c * r_chunk, r_chunk), :] = val
                    if g + 2 < N_DEV * c_chunks:
                        in_dma(g + 2).start()

            w_dma(0, 0, 0).start()

        if not _SKIP_COMM:
            for k in range(1, N_DEV):
                @pl.when((idx == k) & (n == 0))
                def _wait_recv(k=k):
                    recv = pltpu.make_async_remote_copy(
                        src_ref=comm_ref.at[k],
                        dst_ref=comm_ref.at[k],
                        send_sem=send_sems.at[0],
                        recv_sem=recv_sems.at[k],
                        device_id=(my,),
                        device_id_type=pl.DeviceIdType.MESH,
                    )
                    recv.wait_recv()

        @pl.when(lin < last)
        def _prefetch():
            nlin = lin + 1
            nidx = nlin // n_tiles
            nn = nlin % n_tiles
            npar = nlin % 2
            w_dma(nidx, nn, npar).start()
            @pl.when(nidx > 0)
            def _():
                a_dma(nn, npar).start()

        @pl.when(lin >= 1)
        def _obuf_free():
            o_dma(n).wait()

        w_dma(idx, n, par).wait()
        @pl.when(idx > 0)
        def _await_acc():
            a_dma(n, par).wait()

        wtile = wbuf[par].astype(jnp.bfloat16)
        c = 0.7978845608028654
        for mi in range(m_sub):
            row = pl.ds(mi * m_tile, m_tile)
            part = jnp.dot(
                comm_ref[idx, row, :], wtile,
                preferred_element_type=jnp.float32,
            )
            @pl.when(idx == 0)
            def _init(row=row, part=part):
                obuf[row, :] = part
            @pl.when((idx > 0) & (idx < N_DEV - 1))
            def _acc(row=row, part=part):
                obuf[row, :] = abuf[par, row, :] + part
            @pl.when(idx == N_DEV - 1)
            def _fin(row=row, part=part):
                a = abuf[par, row, :] + part
                obuf[row, :] = 0.5 * a * (
                    1.0 + jnp.tanh(c * (a + 0.044715 * a * a * a))
                )

        o_dma(n).start()

        @pl.when(lin == last)
        def _drain():
            o_dma(n).wait()
            if not _SKIP_COMM:
                for d in range(3):
                    off = D_OFF[d]
                    dstd = (my + off) % N_DEV
                    slot = INV_SLOT[off]
                    send = pltpu.make_async_remote_copy(
                        src_ref=x16_scr.at[d],
                        dst_ref=comm_ref.at[slot],
                        send_sem=send_sems.at[d],
                        recv_sem=recv_sems.at[slot],
                        device_id=(dstd,),
                        device_id_type=pl.DeviceIdType.MESH,
                    )
                    send.wait_send()

    out, _ = pl.pallas_call(
        body,
        grid=(N_DEV, n_tiles),
        out_shape=[
            jax.ShapeDtypeStruct((m_blk, n_out), jnp.float32),
            jax.ShapeDtypeStruct((3, m_blk, k_blk), jnp.bfloat16),
        ],
        in_specs=[
            pl.BlockSpec(memory_space=pl.ANY),
            pl.BlockSpec(memory_space=pl.ANY),
        ],
        out_specs=[
            pl.BlockSpec(memory_space=pl.ANY),
            pl.BlockSpec(memory_space=pl.ANY),
        ],
        scratch_shapes=[
            pltpu.VMEM((N_DEV, m_blk, k_blk), jnp.bfloat16),
            pltpu.VMEM((2, k_blk, n_tile), jnp.float32),
            pltpu.VMEM((2, m_blk, n_tile), jnp.float32),
            pltpu.VMEM((m_blk, n_tile), jnp.float32),
            pltpu.VMEM((2, r_chunk, k_blk), jnp.float32),
            pltpu.VMEM((r_chunk, k_blk), jnp.bfloat16),
            pltpu.SemaphoreType.DMA((3,)),
            pltpu.SemaphoreType.DMA((N_DEV,)),
            pltpu.SemaphoreType.DMA((3,)),
            pltpu.SemaphoreType.DMA((2,)),
            pltpu.SemaphoreType.DMA((2,)),
            pltpu.SemaphoreType.DMA,
        ],
        compiler_params=pltpu.CompilerParams(
            dimension_semantics=("arbitrary", "arbitrary"),
            collective_id=None if _SKIP_COMM else 0,
            vmem_limit_bytes=100 * 1024 * 1024,
        ),
    )(x, w_mat)
    return out


# device time: 333709 ns/iter; 1.2559x vs baseline; 1.1591x over previous
import os

import jax
import jax.numpy as jnp
from jax import lax
from jax.experimental import pallas as pl
from jax.experimental.pallas import tpu as pltpu

_SKIP_COMM = os.environ.get("SKIP_COMM") == "1"

N_DEV = 4
OFFS = (0, 1, 3, 2)
INV_SLOT = {1: 2, 2: 3, 3: 1}
SEND_ORDER = (1, 3, 2)


D_OFF = (1, 3, 2, 0)


def kernel(x, w_mat):
    m_total, k_blk = x.shape
    k_total, n_out = w_mat.shape
    m_blk = m_total // N_DEV
    n_tiles = 8
    n_tile = n_out // n_tiles
    m_sub = 4
    m_tile = m_blk // m_sub
    r_chunk = 256
    c_chunks = m_blk // r_chunk

    def body(x_hbm, w_hbm, out_hbm, x16_scr, comm_ref, wbuf, abuf, obuf,
             cbuf, stage, send_sems, recv_sems, conv_sem, w_sems, a_sems,
             o_sems):
        idx = pl.program_id(0)
        n = pl.program_id(1)
        my = lax.axis_index("i")
        lin = idx * n_tiles + n
        par = lin % 2
        last = N_DEV * n_tiles - 1

        def w_dma(i_, n_, p_):
            off_val = jnp.where(i_ == 1, 1, jnp.where(i_ == 2, 3,
                                                      jnp.where(i_ == 3, 2, 0)))
            jj = (my + off_val) % N_DEV
            return pltpu.make_async_copy(
                w_hbm.at[pl.ds(jj * k_blk, k_blk), pl.ds(n_ * n_tile, n_tile)],
                wbuf.at[p_],
                w_sems.at[p_],
            )

        def a_dma(n_, p_):
            return pltpu.make_async_copy(
                out_hbm.at[:, pl.ds(n_ * n_tile, n_tile)],
                abuf.at[p_],
                a_sems.at[p_],
            )

        def o_dma(n_, p_):
            return pltpu.make_async_copy(
                obuf.at[p_],
                out_hbm.at[:, pl.ds(n_ * n_tile, n_tile)],
                o_sems.at[p_],
            )

        @pl.when(lin == 0)
        def _comm():
            if not _SKIP_COMM:
                barrier = pltpu.get_barrier_semaphore()
                for off in (1, 2, 3):
                    pl.semaphore_signal(
                        barrier, inc=1,
                        device_id=((my + off) % N_DEV,),
                        device_id_type=pl.DeviceIdType.MESH,
                    )
                pl.semaphore_wait(barrier, N_DEV - 1)

            def in_dma(g):
                d, c = divmod(g, c_chunks)
                dst = (my + D_OFF[d]) % N_DEV
                return pltpu.make_async_copy(
                    x_hbm.at[pl.ds(dst * m_blk + c * r_chunk, r_chunk), :],
                    cbuf,
                    conv_sem.at[0],
                )

            def conv_out_dma(g):
                d, c = divmod(g, c_chunks)
                return pltpu.make_async_copy(
                    stage,
                    x16_scr.at[d, pl.ds(c * r_chunk, r_chunk), :],
                    conv_sem.at[2],
                )

            in_dma(0).start()
            pending = None
            for g in range(N_DEV * c_chunks):
                d, c = divmod(g, c_chunks)
                in_dma(g).wait()
                val = cbuf[:, :].astype(jnp.bfloat16)
                if d < 3:
                    if pending is not None:
                        conv_out_dma(pending).wait()
                        pending = None
                    stage[:, :] = val
                    if g + 1 < N_DEV * c_chunks:
                        in_dma(g + 1).start()
                    conv_out_dma(g).start()
                    pending = g
                    if c == c_chunks - 1:
                        conv_out_dma(pending).wait()
                        pending = None
                        if not _SKIP_COMM:
                            off = D_OFF[d]
                            dstd = (my + off) % N_DEV
                            slot = INV_SLOT[off]
                            rdma = pltpu.make_async_remote_copy(
                                src_ref=x16_scr.at[d],
                                dst_ref=comm_ref.at[slot],
                                send_sem=send_sems.at[d],
                                recv_sem=recv_sems.at[slot],
                                device_id=(dstd,),
                                device_id_type=pl.DeviceIdType.MESH,
                            )
                            rdma.start()
                else:
                    comm_ref[0, pl.ds(c * r_chunk, r_chunk), :] = val
                    if g + 1 < N_DEV * c_chunks:
                        in_dma(g + 1).start()

            w_dma(0, 0, 0).start()

        if not _SKIP_COMM:
            for k in range(1, N_DEV):
                @pl.when((idx == k) & (n == 0))
                def _wait_recv(k=k):
                    recv = pltpu.make_async_remote_copy(
                        src_ref=comm_ref.at[k],
                        dst_ref=comm_ref.at[k],
                        send_sem=send_sems.at[0],
                        recv_sem=recv_sems.at[k],
                        device_id=(my,),
                        device_id_type=pl.DeviceIdType.MESH,
                    )
                    recv.wait_recv()

        @pl.when(lin < last)
        def _prefetch():
            nlin = lin + 1
            nidx = nlin // n_tiles
            nn = nlin % n_tiles
            npar = nlin % 2
            w_dma(nidx, nn, npar).start()
            @pl.when(nidx > 0)
            def _():
                a_dma(nn, npar).start()

        @pl.when(lin >= 2)
        def _obuf_free():
            o_dma(n, par).wait()

        w_dma(idx, n, par).wait()
        @pl.when(idx > 0)
        def _await_acc():
            a_dma(n, par).wait()

        wtile = wbuf[par].astype(jnp.bfloat16)
        c = 0.7978845608028654
        for mi in range(m_sub):
            row = pl.ds(mi * m_tile, m_tile)
            part = jnp.dot(
                comm_ref[idx, row, :], wtile,
                preferred_element_type=jnp.float32,
            )
            @pl.when(idx == 0)
            def _init(row=row, part=part):
                obuf[par, row, :] = part
            @pl.when((idx > 0) & (idx < N_DEV - 1))
            def _acc(row=row, part=part):
                obuf[par, row, :] = abuf[par, row, :] + part
            @pl.when(idx == N_DEV - 1)
            def _fin(row=row, part=part):
                a = abuf[par, row, :] + part
                obuf[par, row, :] = 0.5 * a * (
                    1.0 + jnp.tanh(c * (a + 0.044715 * a * a * a))
                )

        o_dma(n, par).start()

        @pl.when(lin == last)
        def _drain():
            o_dma(n, 1 - par).wait()
            o_dma(n, par).wait()
            if not _SKIP_COMM:
                for d in range(3):
                    off = D_OFF[d]
                    dstd = (my + off) % N_DEV
                    slot = INV_SLOT[off]
                    send = pltpu.make_async_remote_copy(
                        src_ref=x16_scr.at[d],
                        dst_ref=comm_ref.at[slot],
                        send_sem=send_sems.at[d],
                        recv_sem=recv_sems.at[slot],
                        device_id=(dstd,),
                        device_id_type=pl.DeviceIdType.MESH,
                    )
                    send.wait_send()

    out, _ = pl.pallas_call(
        body,
        grid=(N_DEV, n_tiles),
        out_shape=[
            jax.ShapeDtypeStruct((m_blk, n_out), jnp.float32),
            jax.ShapeDtypeStruct((3, m_blk, k_blk), jnp.bfloat16),
        ],
        in_specs=[
            pl.BlockSpec(memory_space=pl.ANY),
            pl.BlockSpec(memory_space=pl.ANY),
        ],
        out_specs=[
            pl.BlockSpec(memory_space=pl.ANY),
            pl.BlockSpec(memory_space=pl.ANY),
        ],
        scratch_shapes=[
            pltpu.VMEM((N_DEV, m_blk, k_blk), jnp.bfloat16),
            pltpu.VMEM((2, k_blk, n_tile), jnp.float32),
            pltpu.VMEM((2, m_blk, n_tile), jnp.float32),
            pltpu.VMEM((2, m_blk, n_tile), jnp.float32),
            pltpu.VMEM((r_chunk, k_blk), jnp.float32),
            pltpu.VMEM((r_chunk, k_blk), jnp.bfloat16),
            pltpu.SemaphoreType.DMA((3,)),
            pltpu.SemaphoreType.DMA((N_DEV,)),
            pltpu.SemaphoreType.DMA((3,)),
            pltpu.SemaphoreType.DMA((2,)),
            pltpu.SemaphoreType.DMA((2,)),
            pltpu.SemaphoreType.DMA((2,)),
        ],
        compiler_params=pltpu.CompilerParams(
            dimension_semantics=("arbitrary", "arbitrary"),
            collective_id=None if _SKIP_COMM else 0,
            vmem_limit_bytes=100 * 1024 * 1024,
        ),
    )(x, w_mat)
    return out
